# Initial kernel scaffold; baseline (speedup 1.0000x reference)
import functools

import jax
import jax.numpy as jnp
from jax import lax
from jax.experimental import pallas as pl
from jax.experimental.pallas import tpu as pltpu
from jax.experimental.pallas import tpu_sc as plsc

N = 10000
E = 320000
F_IN = 128
H = 32

NC = 2
NS = 16
NW = NC * NS
CHUNK = 128
KCH = 80
EPW = KCH * CHUNK
EPAD = NW * EPW
NPAD = 10016
ROWS_PER_TILE = NPAD // NS


_MESH = plsc.VectorSubcoreMesh(core_axis_name="c", subcore_axis_name="s")


@functools.partial(
    pl.kernel,
    out_type=jax.ShapeDtypeStruct((NC, NPAD, 16), jnp.float32),
    mesh=_MESH,
    scratch_types=[
        pltpu.VMEM((KCH, CHUNK), jnp.int32),
        pltpu.VMEM((CHUNK, 16), jnp.float32),
        pltpu.VMEM_SHARED((NPAD, 16), jnp.float32),
    ],
)
def _sc_degree(dst_hbm, ones_hbm, zeros_hbm, out_hbm, dst_v, ones_v, deg_s):
    cid = lax.axis_index("c")
    sid = lax.axis_index("s")
    wid = cid * NS + sid

    pltpu.sync_copy(dst_hbm.at[wid], dst_v)
    pltpu.sync_copy(ones_hbm, ones_v)
    pltpu.sync_copy(
        zeros_hbm.at[pl.ds(sid * ROWS_PER_TILE, ROWS_PER_TILE)],
        deg_s.at[pl.ds(sid * ROWS_PER_TILE, ROWS_PER_TILE)],
    )
    plsc.subcore_barrier()

    def body(j, carry):
        pltpu.sync_copy(ones_v, deg_s.at[dst_v.at[j]], add=True)
        return carry

    lax.fori_loop(0, KCH, body, 0)
    plsc.subcore_barrier()

    pltpu.sync_copy(
        deg_s.at[pl.ds(sid * ROWS_PER_TILE, ROWS_PER_TILE)],
        out_hbm.at[cid, pl.ds(sid * ROWS_PER_TILE, ROWS_PER_TILE)],
    )


@functools.partial(
    pl.kernel,
    out_type=jax.ShapeDtypeStruct((NC, NPAD, H), jnp.float32),
    mesh=_MESH,
    scratch_types=[
        pltpu.VMEM((KCH, CHUNK), jnp.int32),
        pltpu.VMEM((KCH, CHUNK), jnp.int32),
        pltpu.VMEM((CHUNK, H), jnp.float32),
        pltpu.VMEM_SHARED((NPAD, H), jnp.float32),
        pltpu.SemaphoreType.DMA,
    ],
)
def _sc_aggregate(table_hbm, src_hbm, dst_hbm, zeros_hbm, out_hbm,
                  src_v, dst_v, rows_v, acc_s, gsem):
    cid = lax.axis_index("c")
    sid = lax.axis_index("s")
    wid = cid * NS + sid

    pltpu.sync_copy(src_hbm.at[wid], src_v)
    pltpu.sync_copy(dst_hbm.at[wid], dst_v)
    pltpu.sync_copy(
        zeros_hbm.at[pl.ds(sid * ROWS_PER_TILE, ROWS_PER_TILE)],
        acc_s.at[pl.ds(sid * ROWS_PER_TILE, ROWS_PER_TILE)],
    )
    plsc.subcore_barrier()

    def body(j, carry):
        pltpu.async_copy(table_hbm.at[src_v.at[j]], rows_v, gsem).wait()
        pltpu.sync_copy(rows_v, acc_s.at[dst_v.at[j]], add=True)
        return carry

    lax.fori_loop(0, KCH, body, 0)
    plsc.subcore_barrier()

    pltpu.sync_copy(
        acc_s.at[pl.ds(sid * ROWS_PER_TILE, ROWS_PER_TILE)],
        out_hbm.at[cid, pl.ds(sid * ROWS_PER_TILE, ROWS_PER_TILE)],
    )


def _tc_prep_body(x_ref, w1_ref, degp_ref, h1p_ref, dinv_ref):
    deg = 1.0 + degp_ref[0, :, 0:1] + degp_ref[1, :, 0:1]
    dinv = lax.rsqrt(deg)
    h = jnp.dot(x_ref[...], w1_ref[...], preferred_element_type=jnp.float32)
    h1p_ref[...] = h * dinv
    dinv_ref[...] = dinv


def _tc_prep(x, w1, degp):
    return pl.pallas_call(
        _tc_prep_body,
        out_shape=(
            jax.ShapeDtypeStruct((NPAD, H), jnp.float32),
            jax.ShapeDtypeStruct((NPAD, 1), jnp.float32),
        ),
    )(x, w1, degp)


def _tc_mid_body(accp_ref, h1p_ref, dinv_ref, w2_ref, b1_ref, h2p_ref):
    dinv = dinv_ref[...]
    acc = accp_ref[0] + accp_ref[1] + h1p_ref[...]
    h1 = jnp.maximum(acc * dinv + b1_ref[...], 0.0)
    g = jnp.dot(h1, w2_ref[...], preferred_element_type=jnp.float32)
    h2p_ref[...] = g * dinv


def _tc_mid(accp, h1p, dinv, w2, b1):
    return pl.pallas_call(
        _tc_mid_body,
        out_shape=jax.ShapeDtypeStruct((NPAD, H), jnp.float32),
    )(accp, h1p, dinv, w2, b1.reshape(1, H))


def _tc_final_body(accp_ref, h2p_ref, dinv_ref, w3_ref, b2_ref, b3_ref, out_ref):
    dinv = dinv_ref[...]
    acc = accp_ref[0] + accp_ref[1] + h2p_ref[...]
    h2 = jnp.maximum(acc * dinv + b2_ref[...], 0.0)
    out_ref[...] = jnp.dot(h2, w3_ref[...], preferred_element_type=jnp.float32) + b3_ref[...]


def _tc_final(accp, h2p, dinv, w3, b2, b3):
    return pl.pallas_call(
        _tc_final_body,
        out_shape=jax.ShapeDtypeStruct((NPAD, 1), jnp.float32),
    )(accp, h2p, dinv, w3, b2.reshape(1, H), b3.reshape(1, 1))


@jax.jit
def kernel(x, edge_index, W1, b1, W2, b2, W3, b3):
    pad_e = EPAD - E
    src = jnp.concatenate(
        [edge_index[0], jnp.zeros((pad_e,), jnp.int32)]).reshape(NW, KCH, CHUNK)
    dst = jnp.concatenate(
        [edge_index[1], jnp.full((pad_e,), N, jnp.int32)]).reshape(NW, KCH, CHUNK)

    x_pad = jnp.pad(x, ((0, NPAD - N), (0, 0)))
    ones_rows = jnp.ones((CHUNK, 16), jnp.float32)
    zeros16 = jnp.zeros((NPAD, 16), jnp.float32)
    zeros_h = jnp.zeros((NPAD, H), jnp.float32)

    degp = _sc_degree(dst, ones_rows, zeros16)
    h1p, dinv = _tc_prep(x_pad, W1, degp)
    acc1 = _sc_aggregate(h1p, src, dst, zeros_h)
    h2p = _tc_mid(acc1, h1p, dinv, W2, b1)
    acc2 = _sc_aggregate(h2p, src, dst, zeros_h)
    out = _tc_final(acc2, h2p, dinv, W3, b2, b3)
    return out[:N]


# speedup vs baseline: 22.5972x; 22.5972x over previous
import functools

import jax
import jax.numpy as jnp
from jax import lax
from jax.experimental import pallas as pl
from jax.experimental.pallas import tpu as pltpu
from jax.experimental.pallas import tpu_sc as plsc

N = 10000
E = 320000
F_IN = 128
H = 32

NC = 2
NS = 16
NW = NC * NS
CHUNK = 128
KCH = 80
EPW = KCH * CHUNK
EPAD = NW * EPW
NPAD = 10112
ROWS_PER_TILE = NPAD // NS


_MESH = plsc.VectorSubcoreMesh(core_axis_name="c", subcore_axis_name="s")
_SC_PARAMS = pltpu.CompilerParams(use_tc_tiling_on_sc=False)


@functools.partial(
    pl.kernel,
    out_type=jax.ShapeDtypeStruct((NC, NPAD, 16), jnp.float32),
    mesh=_MESH,
    compiler_params=_SC_PARAMS,
    scratch_types=[
        pltpu.VMEM((KCH, CHUNK), jnp.int32),
        pltpu.VMEM((CHUNK, 16), jnp.float32),
        pltpu.VMEM_SHARED((NPAD, 16), jnp.float32),
    ],
)
def _sc_degree(dst_hbm, ones_hbm, zeros_hbm, out_hbm, dst_v, ones_v, deg_s):
    cid = lax.axis_index("c")
    sid = lax.axis_index("s")
    wid = cid * NS + sid

    pltpu.sync_copy(dst_hbm.at[wid], dst_v)
    pltpu.sync_copy(ones_hbm, ones_v)
    pltpu.sync_copy(
        zeros_hbm.at[pl.ds(sid * ROWS_PER_TILE, ROWS_PER_TILE)],
        deg_s.at[pl.ds(sid * ROWS_PER_TILE, ROWS_PER_TILE)],
    )
    plsc.subcore_barrier()

    def body(j, carry):
        pltpu.sync_copy(ones_v, deg_s.at[dst_v.at[j]], add=True)
        return carry

    lax.fori_loop(0, KCH, body, 0)
    plsc.subcore_barrier()

    pltpu.sync_copy(
        deg_s.at[pl.ds(sid * ROWS_PER_TILE, ROWS_PER_TILE)],
        out_hbm.at[cid, pl.ds(sid * ROWS_PER_TILE, ROWS_PER_TILE)],
    )


@functools.partial(
    pl.kernel,
    out_type=jax.ShapeDtypeStruct((NC, NPAD, H), jnp.float32),
    mesh=_MESH,
    compiler_params=_SC_PARAMS,
    scratch_types=[
        pltpu.VMEM((KCH, CHUNK), jnp.int32),
        pltpu.VMEM((KCH, CHUNK), jnp.int32),
        pltpu.VMEM((CHUNK, H), jnp.float32),
        pltpu.VMEM_SHARED((NPAD, H), jnp.float32),
        pltpu.SemaphoreType.DMA,
    ],
)
def _sc_aggregate(table_hbm, src_hbm, dst_hbm, zeros_hbm, out_hbm,
                  src_v, dst_v, rows_v, acc_s, gsem):
    cid = lax.axis_index("c")
    sid = lax.axis_index("s")
    wid = cid * NS + sid

    pltpu.sync_copy(src_hbm.at[wid], src_v)
    pltpu.sync_copy(dst_hbm.at[wid], dst_v)
    pltpu.sync_copy(
        zeros_hbm.at[pl.ds(sid * ROWS_PER_TILE, ROWS_PER_TILE)],
        acc_s.at[pl.ds(sid * ROWS_PER_TILE, ROWS_PER_TILE)],
    )
    plsc.subcore_barrier()

    def body(j, carry):
        pltpu.async_copy(table_hbm.at[src_v.at[j]], rows_v, gsem).wait()
        pltpu.sync_copy(rows_v, acc_s.at[dst_v.at[j]], add=True)
        return carry

    lax.fori_loop(0, KCH, body, 0)
    plsc.subcore_barrier()

    pltpu.sync_copy(
        acc_s.at[pl.ds(sid * ROWS_PER_TILE, ROWS_PER_TILE)],
        out_hbm.at[cid, pl.ds(sid * ROWS_PER_TILE, ROWS_PER_TILE)],
    )


def _tc_prep_body(x_ref, w1_ref, degp_ref, h1p_ref, dinv_ref):
    deg = 1.0 + degp_ref[0, :, 0:1] + degp_ref[1, :, 0:1]
    dinv = lax.rsqrt(deg)
    h = jnp.dot(x_ref[...], w1_ref[...], preferred_element_type=jnp.float32)
    h1p_ref[...] = h * dinv
    dinv_ref[...] = dinv


def _tc_prep(x, w1, degp):
    return pl.pallas_call(
        _tc_prep_body,
        out_shape=(
            jax.ShapeDtypeStruct((NPAD, H), jnp.float32),
            jax.ShapeDtypeStruct((NPAD, 1), jnp.float32),
        ),
    )(x, w1, degp)


def _tc_mid_body(accp_ref, h1p_ref, dinv_ref, w2_ref, b1_ref, h2p_ref):
    dinv = dinv_ref[...]
    acc = accp_ref[0] + accp_ref[1] + h1p_ref[...]
    h1 = jnp.maximum(acc * dinv + b1_ref[...], 0.0)
    g = jnp.dot(h1, w2_ref[...], preferred_element_type=jnp.float32)
    h2p_ref[...] = g * dinv


def _tc_mid(accp, h1p, dinv, w2, b1):
    return pl.pallas_call(
        _tc_mid_body,
        out_shape=jax.ShapeDtypeStruct((NPAD, H), jnp.float32),
    )(accp, h1p, dinv, w2, b1.reshape(1, H))


def _tc_final_body(accp_ref, h2p_ref, dinv_ref, w3_ref, b2_ref, b3_ref, out_ref):
    dinv = dinv_ref[...]
    acc = accp_ref[0] + accp_ref[1] + h2p_ref[...]
    h2 = jnp.maximum(acc * dinv + b2_ref[...], 0.0)
    out_ref[...] = jnp.dot(h2, w3_ref[...], preferred_element_type=jnp.float32) + b3_ref[...]


def _tc_final(accp, h2p, dinv, w3, b2, b3):
    return pl.pallas_call(
        _tc_final_body,
        out_shape=jax.ShapeDtypeStruct((NPAD, 1), jnp.float32),
    )(accp, h2p, dinv, w3, b2.reshape(1, H), b3.reshape(1, 1))


@jax.jit
def kernel(x, edge_index, W1, b1, W2, b2, W3, b3):
    pad_e = EPAD - E
    src = jnp.concatenate(
        [edge_index[0], jnp.zeros((pad_e,), jnp.int32)]).reshape(NW, KCH, CHUNK)
    dst = jnp.concatenate(
        [edge_index[1], jnp.full((pad_e,), N, jnp.int32)]).reshape(NW, KCH, CHUNK)

    x_pad = jnp.pad(x, ((0, NPAD - N), (0, 0)))
    ones_rows = jnp.ones((CHUNK, 16), jnp.float32)
    zeros16 = jnp.zeros((NPAD, 16), jnp.float32)
    zeros_h = jnp.zeros((NPAD, H), jnp.float32)

    degp = _sc_degree(dst, ones_rows, zeros16)
    h1p, dinv = _tc_prep(x_pad, W1, degp)
    acc1 = _sc_aggregate(h1p, src, dst, zeros_h)
    h2p = _tc_mid(acc1, h1p, dinv, W2, b1)
    acc2 = _sc_aggregate(h2p, src, dst, zeros_h)
    out = _tc_final(acc2, h2p, dinv, W3, b2, b3)
    return out[:N]


# speedup vs baseline: 27.3839x; 1.2118x over previous
import functools

import jax
import jax.numpy as jnp
from jax import lax
from jax.experimental import pallas as pl
from jax.experimental.pallas import tpu as pltpu
from jax.experimental.pallas import tpu_sc as plsc

N = 10000
E = 320000
F_IN = 128
H = 32

NC = 2
NS = 16
NW = NC * NS
CHUNK = 128
KCH = 80
GRP = 8
NG = KCH // GRP
EPW = KCH * CHUNK
EPAD = NW * EPW
NPAD = 10112
ROWS_PER_TILE = NPAD // NS


_MESH = plsc.VectorSubcoreMesh(core_axis_name="c", subcore_axis_name="s")
_SC_PARAMS = pltpu.CompilerParams(use_tc_tiling_on_sc=False)


@functools.partial(
    pl.kernel,
    out_type=jax.ShapeDtypeStruct((NC, NPAD, 16), jnp.float32),
    mesh=_MESH,
    compiler_params=_SC_PARAMS,
    scratch_types=[
        pltpu.VMEM((KCH, CHUNK), jnp.int32),
        pltpu.VMEM((CHUNK, 16), jnp.float32),
        pltpu.VMEM_SHARED((NPAD, 16), jnp.float32),
        pltpu.SemaphoreType.DMA,
    ],
)
def _sc_degree(dst_hbm, ones_hbm, zeros_hbm, out_hbm, dst_v, ones_v, deg_s, ssem):
    cid = lax.axis_index("c")
    sid = lax.axis_index("s")
    wid = cid * NS + sid

    pltpu.sync_copy(dst_hbm.at[wid], dst_v)
    pltpu.sync_copy(ones_hbm, ones_v)
    pltpu.sync_copy(
        zeros_hbm.at[pl.ds(sid * ROWS_PER_TILE, ROWS_PER_TILE)],
        deg_s.at[pl.ds(sid * ROWS_PER_TILE, ROWS_PER_TILE)],
    )
    plsc.subcore_barrier()

    def fire(j, carry):
        pltpu.async_copy(ones_v, deg_s.at[dst_v.at[j]], ssem, add=True)
        return carry

    def drain(j, carry):
        pltpu.make_async_copy(ones_v, deg_s.at[dst_v.at[j]], ssem).wait()
        return carry

    lax.fori_loop(0, KCH, fire, 0)
    lax.fori_loop(0, KCH, drain, 0)
    plsc.subcore_barrier()

    pltpu.sync_copy(
        deg_s.at[pl.ds(sid * ROWS_PER_TILE, ROWS_PER_TILE)],
        out_hbm.at[cid, pl.ds(sid * ROWS_PER_TILE, ROWS_PER_TILE)],
    )


@functools.partial(
    pl.kernel,
    out_type=jax.ShapeDtypeStruct((NC, NPAD, H), jnp.float32),
    mesh=_MESH,
    compiler_params=_SC_PARAMS,
    scratch_types=[
        pltpu.VMEM((KCH, CHUNK), jnp.int32),
        pltpu.VMEM((KCH, CHUNK), jnp.int32),
        pltpu.VMEM((2, GRP, CHUNK, H), jnp.float32),
        pltpu.VMEM_SHARED((NPAD, H), jnp.float32),
        pltpu.SemaphoreType.DMA,
        pltpu.SemaphoreType.DMA,
    ],
)
def _sc_aggregate(table_hbm, src_hbm, dst_hbm, zeros_hbm, out_hbm,
                  src_v, dst_v, rows_v, acc_s, gsem, ssem):
    cid = lax.axis_index("c")
    sid = lax.axis_index("s")
    wid = cid * NS + sid

    pltpu.sync_copy(src_hbm.at[wid], src_v)
    pltpu.sync_copy(dst_hbm.at[wid], dst_v)
    pltpu.sync_copy(
        zeros_hbm.at[pl.ds(sid * ROWS_PER_TILE, ROWS_PER_TILE)],
        acc_s.at[pl.ds(sid * ROWS_PER_TILE, ROWS_PER_TILE)],
    )
    plsc.subcore_barrier()

    def fire_gathers(g, p):
        for b in range(GRP):
            pltpu.async_copy(table_hbm.at[src_v.at[g * GRP + b]],
                             rows_v.at[p, b], gsem)

    def drain_gathers(p):
        for b in range(GRP):
            pltpu.make_async_copy(table_hbm.at[src_v.at[0]],
                                  rows_v.at[p, b], gsem).wait()

    def fire_scatters(g, p):
        for b in range(GRP):
            pltpu.async_copy(rows_v.at[p, b],
                             acc_s.at[dst_v.at[g * GRP + b]], ssem, add=True)

    def drain_scatters(g, p):
        for b in range(GRP):
            pltpu.make_async_copy(rows_v.at[p, b],
                                  acc_s.at[dst_v.at[g * GRP + b]], ssem).wait()

    fire_gathers(0, 0)
    fire_gathers(1, 1)

    def body(k, carry):
        g0 = 2 * k
        g1 = g0 + 1
        drain_gathers(0)
        fire_scatters(g0, 0)
        drain_gathers(1)
        drain_scatters(g0, 0)

        @pl.when(g0 + 2 < NG)
        def _():
            fire_gathers(g0 + 2, 0)

        fire_scatters(g1, 1)
        drain_scatters(g1, 1)

        @pl.when(g1 + 2 < NG)
        def _():
            fire_gathers(g1 + 2, 1)

        return carry

    lax.fori_loop(0, NG // 2, body, 0)
    plsc.subcore_barrier()

    pltpu.sync_copy(
        acc_s.at[pl.ds(sid * ROWS_PER_TILE, ROWS_PER_TILE)],
        out_hbm.at[cid, pl.ds(sid * ROWS_PER_TILE, ROWS_PER_TILE)],
    )


def _tc_prep_body(x_ref, w1_ref, degp_ref, h1p_ref, dinv_ref):
    deg = 1.0 + degp_ref[0, :, 0:1] + degp_ref[1, :, 0:1]
    dinv = lax.rsqrt(deg)
    h = jnp.dot(x_ref[...], w1_ref[...], preferred_element_type=jnp.float32)
    h1p_ref[...] = h * dinv
    dinv_ref[...] = dinv


def _tc_prep(x, w1, degp):
    return pl.pallas_call(
        _tc_prep_body,
        out_shape=(
            jax.ShapeDtypeStruct((NPAD, H), jnp.float32),
            jax.ShapeDtypeStruct((NPAD, 1), jnp.float32),
        ),
    )(x, w1, degp)


def _tc_mid_body(accp_ref, h1p_ref, dinv_ref, w2_ref, b1_ref, h2p_ref):
    dinv = dinv_ref[...]
    acc = accp_ref[0] + accp_ref[1] + h1p_ref[...]
    h1 = jnp.maximum(acc * dinv + b1_ref[...], 0.0)
    g = jnp.dot(h1, w2_ref[...], preferred_element_type=jnp.float32)
    h2p_ref[...] = g * dinv


def _tc_mid(accp, h1p, dinv, w2, b1):
    return pl.pallas_call(
        _tc_mid_body,
        out_shape=jax.ShapeDtypeStruct((NPAD, H), jnp.float32),
    )(accp, h1p, dinv, w2, b1.reshape(1, H))


def _tc_final_body(accp_ref, h2p_ref, dinv_ref, w3_ref, b2_ref, b3_ref, out_ref):
    dinv = dinv_ref[...]
    acc = accp_ref[0] + accp_ref[1] + h2p_ref[...]
    h2 = jnp.maximum(acc * dinv + b2_ref[...], 0.0)
    out_ref[...] = jnp.dot(h2, w3_ref[...], preferred_element_type=jnp.float32) + b3_ref[...]


def _tc_final(accp, h2p, dinv, w3, b2, b3):
    return pl.pallas_call(
        _tc_final_body,
        out_shape=jax.ShapeDtypeStruct((NPAD, 1), jnp.float32),
    )(accp, h2p, dinv, w3, b2.reshape(1, H), b3.reshape(1, 1))


@jax.jit
def kernel(x, edge_index, W1, b1, W2, b2, W3, b3):
    pad_e = EPAD - E
    src = jnp.concatenate(
        [edge_index[0], jnp.zeros((pad_e,), jnp.int32)]).reshape(NW, KCH, CHUNK)
    dst = jnp.concatenate(
        [edge_index[1], jnp.full((pad_e,), N, jnp.int32)]).reshape(NW, KCH, CHUNK)

    x_pad = jnp.pad(x, ((0, NPAD - N), (0, 0)))
    ones_rows = jnp.ones((CHUNK, 16), jnp.float32)
    zeros16 = jnp.zeros((NPAD, 16), jnp.float32)
    zeros_h = jnp.zeros((NPAD, H), jnp.float32)

    degp = _sc_degree(dst, ones_rows, zeros16)
    h1p, dinv = _tc_prep(x_pad, W1, degp)
    acc1 = _sc_aggregate(h1p, src, dst, zeros_h)
    h2p = _tc_mid(acc1, h1p, dinv, W2, b1)
    acc2 = _sc_aggregate(h2p, src, dst, zeros_h)
    out = _tc_final(acc2, h2p, dinv, W3, b2, b3)
    return out[:N]
